# Initial kernel scaffold; baseline (speedup 1.0000x reference)
import jax
import jax.numpy as jnp
from jax.experimental import pallas as pl


def kernel(x, edge_index, W1, b1, W2, b2, W3, b3, Wc, bc):
    raise NotImplementedError("write your pallas kernel here")


# speedup vs baseline: 11.0302x; 11.0302x over previous
import functools

import jax
import jax.numpy as jnp
from jax import lax
from jax.experimental import pallas as pl
from jax.experimental.pallas import tpu as pltpu
from jax.experimental.pallas import tpu_sc as plsc

N_NODES = 10000
N_PAD = 10240
E = 320000
NC, NS = 2, 16
CHUNK = 128
SUP = 8
ROWS = 2560
E_PAD = ROWS * CHUNK
ROWS_PER_CORE = ROWS // NC
ROWS_PER_TILE = ROWS_PER_CORE // NS
OUTER = ROWS_PER_TILE // SUP
STRIPE = N_PAD // NS
RB = 1024
GRID = N_PAD // RB

_mesh = plsc.VectorSubcoreMesh(core_axis_name="c", subcore_axis_name="s")


def _zero_vmem_2d(ref, rows, cols):
    def row(i, carry):
        for c0 in range(0, cols, 16):
            ref[i, pl.ds(c0, 16)] = jnp.zeros((16,), jnp.float32)
        return carry
    lax.fori_loop(0, rows, row, 0)


@functools.partial(
    pl.kernel,
    out_type=jax.ShapeDtypeStruct((NC, N_PAD), jnp.float32),
    mesh=_mesh,
    scratch_types=[
        pltpu.VMEM_SHARED((N_PAD,), jnp.float32),
        pltpu.VMEM((SUP, CHUNK), jnp.int32),
        pltpu.VMEM((CHUNK,), jnp.float32),
        pltpu.VMEM((STRIPE,), jnp.float32),
    ],
)
def _sc_degree(dst_hbm, out_hbm, acc, didx, ones_v, zstripe):
    c = lax.axis_index("c")
    s = lax.axis_index("s")

    def fill(i, carry):
        ones_v[pl.ds(i * 16, 16)] = jnp.ones((16,), jnp.float32)
        return carry
    lax.fori_loop(0, CHUNK // 16, fill, 0)

    def zfill(i, carry):
        zstripe[pl.ds(i * 16, 16)] = jnp.zeros((16,), jnp.float32)
        return carry
    lax.fori_loop(0, STRIPE // 16, zfill, 0)
    pltpu.sync_copy(zstripe, acc.at[pl.ds(s * STRIPE, STRIPE)])
    plsc.subcore_barrier()

    base_row = c * ROWS_PER_CORE + s * ROWS_PER_TILE

    def outer(o, carry):
        pltpu.sync_copy(dst_hbm.at[pl.ds(base_row + o * SUP, SUP)], didx)
        for j in range(SUP):
            pltpu.sync_copy(ones_v, acc.at[didx.at[j]], add=True)
        return carry
    lax.fori_loop(0, OUTER, outer, 0)
    plsc.subcore_barrier()
    pltpu.sync_copy(acc.at[pl.ds(s * STRIPE, STRIPE)],
                    out_hbm.at[c, pl.ds(s * STRIPE, STRIPE)])


def _make_agg(C):
    @functools.partial(
        pl.kernel,
        out_type=jax.ShapeDtypeStruct((NC, N_PAD, C), jnp.float32),
        mesh=_mesh,
        scratch_types=[
            pltpu.VMEM_SHARED((N_PAD, C), jnp.float32),
            pltpu.VMEM((SUP, CHUNK), jnp.int32),
            pltpu.VMEM((SUP, CHUNK), jnp.int32),
            pltpu.VMEM((CHUNK, C), jnp.float32),
            pltpu.VMEM((CHUNK, C), jnp.float32),
            pltpu.SemaphoreType.DMA,
            pltpu.SemaphoreType.DMA,
        ],
        compiler_params=pltpu.CompilerParams(use_tc_tiling_on_sc=False),
    )
    def agg(table_hbm, src_hbm, dst_hbm, out_hbm,
            acc, sidx, didx, bufa, bufb, sema, semb):
        c = lax.axis_index("c")
        s = lax.axis_index("s")

        _zero_vmem_2d(bufa, CHUNK, C)
        for k in range(STRIPE // CHUNK):
            pltpu.sync_copy(bufa, acc.at[pl.ds(s * STRIPE + k * CHUNK, CHUNK)])
        plsc.subcore_barrier()

        base_row = c * ROWS_PER_CORE + s * ROWS_PER_TILE

        def outer(o, carry):
            r0 = base_row + o * SUP
            pltpu.sync_copy(src_hbm.at[pl.ds(r0, SUP)], sidx)
            pltpu.sync_copy(dst_hbm.at[pl.ds(r0, SUP)], didx)
            cp = pltpu.async_copy(table_hbm.at[sidx.at[0]], bufa, sema)
            for j in range(SUP):
                buf_cur = bufa if j % 2 == 0 else bufb
                buf_nxt = bufb if j % 2 == 0 else bufa
                cp.wait()
                if j + 1 < SUP:
                    sem_nxt = semb if j % 2 == 0 else sema
                    cp = pltpu.async_copy(
                        table_hbm.at[sidx.at[j + 1]], buf_nxt, sem_nxt)
                pltpu.sync_copy(buf_cur, acc.at[didx.at[j]], add=True)
            return carry
        lax.fori_loop(0, OUTER, outer, 0)
        plsc.subcore_barrier()
        for k in range(STRIPE // CHUNK):
            r = s * STRIPE + k * CHUNK
            pltpu.sync_copy(acc.at[pl.ds(r, CHUNK)],
                            out_hbm.at[c, pl.ds(r, CHUNK)])
    return agg


_agg = {C: _make_agg(C) for C in (128, 64, 32)}


def _tc_first_body(x_ref, w_ref, d0_ref, d1_ref, hs_ref, dinv_ref):
    deg = d0_ref[0] + d1_ref[0] + 1.0
    dinv = lax.rsqrt(deg)
    h = jnp.dot(x_ref[...], w_ref[...], preferred_element_type=jnp.float32)
    hs_ref[...] = dinv * h
    dinv_ref[...] = dinv


def _tc_first(xp, W1, D):
    return pl.pallas_call(
        _tc_first_body,
        grid=(GRID,),
        in_specs=[
            pl.BlockSpec((RB, 128), lambda i: (i, 0)),
            pl.BlockSpec((128, 128), lambda i: (0, 0)),
            pl.BlockSpec((1, RB, 1), lambda i: (0, i, 0)),
            pl.BlockSpec((1, RB, 1), lambda i: (1, i, 0)),
        ],
        out_specs=[
            pl.BlockSpec((RB, 128), lambda i: (i, 0)),
            pl.BlockSpec((RB, 1), lambda i: (i, 0)),
        ],
        out_shape=[
            jax.ShapeDtypeStruct((N_PAD, 128), jnp.float32),
            jax.ShapeDtypeStruct((N_PAD, 1), jnp.float32),
        ],
    )(xp, W1, D, D)


def _tc_mid_body(s0_ref, s1_ref, hs_ref, dinv_ref, b_ref, w_ref, o_ref):
    dinv = dinv_ref[...]
    z = dinv * (s0_ref[0] + s1_ref[0] + hs_ref[...]) + b_ref[...]
    ht = jnp.tanh(z)
    o_ref[...] = dinv * jnp.dot(ht, w_ref[...],
                                preferred_element_type=jnp.float32)


def _tc_mid(S, hs, dinv, b, W, Cin, Cout):
    return pl.pallas_call(
        _tc_mid_body,
        grid=(GRID,),
        in_specs=[
            pl.BlockSpec((1, RB, Cin), lambda i: (0, i, 0)),
            pl.BlockSpec((1, RB, Cin), lambda i: (1, i, 0)),
            pl.BlockSpec((RB, Cin), lambda i: (i, 0)),
            pl.BlockSpec((RB, 1), lambda i: (i, 0)),
            pl.BlockSpec((1, Cin), lambda i: (0, 0)),
            pl.BlockSpec((Cin, Cout), lambda i: (0, 0)),
        ],
        out_specs=pl.BlockSpec((RB, Cout), lambda i: (i, 0)),
        out_shape=jax.ShapeDtypeStruct((N_PAD, Cout), jnp.float32),
    )(S, S, hs, dinv, b, W)


def _tc_last_body(s0_ref, s1_ref, hs_ref, dinv_ref, b_ref, wc_ref, bc_ref,
                  out_ref, h_ref):
    dinv = dinv_ref[...]
    z = dinv * (s0_ref[0] + s1_ref[0] + hs_ref[...]) + b_ref[...]
    ht = jnp.tanh(z)
    h_ref[...] = ht
    out_ref[...] = (jnp.dot(ht, wc_ref[...],
                            preferred_element_type=jnp.float32) + bc_ref[...])


def _tc_last(S, hs, dinv, b3, Wc, bc):
    return pl.pallas_call(
        _tc_last_body,
        grid=(GRID,),
        in_specs=[
            pl.BlockSpec((1, RB, 32), lambda i: (0, i, 0)),
            pl.BlockSpec((1, RB, 32), lambda i: (1, i, 0)),
            pl.BlockSpec((RB, 32), lambda i: (i, 0)),
            pl.BlockSpec((RB, 1), lambda i: (i, 0)),
            pl.BlockSpec((1, 32), lambda i: (0, 0)),
            pl.BlockSpec((32, 40), lambda i: (0, 0)),
            pl.BlockSpec((1, 40), lambda i: (0, 0)),
        ],
        out_specs=[
            pl.BlockSpec((RB, 40), lambda i: (i, 0)),
            pl.BlockSpec((RB, 32), lambda i: (i, 0)),
        ],
        out_shape=[
            jax.ShapeDtypeStruct((N_PAD, 40), jnp.float32),
            jax.ShapeDtypeStruct((N_PAD, 32), jnp.float32),
        ],
    )(S, S, hs, dinv, b3, Wc, bc)


def kernel(x, edge_index, W1, b1, W2, b2, W3, b3, Wc, bc):
    src = edge_index[0].astype(jnp.int32)
    dst = edge_index[1].astype(jnp.int32)
    pad = jnp.full((E_PAD - E,), N_NODES, jnp.int32)
    src2d = jnp.concatenate([src, pad]).reshape(ROWS, CHUNK)
    dst2d = jnp.concatenate([dst, pad]).reshape(ROWS, CHUNK)
    xp = jnp.pad(x, ((0, N_PAD - N_NODES), (0, 0)))

    D = _sc_degree(dst2d).reshape(NC, N_PAD, 1)
    hs1, dinv = _tc_first(xp, W1, D)
    S1 = _agg[128](hs1, src2d, dst2d)
    hs2 = _tc_mid(S1, hs1, dinv, b1.reshape(1, -1), W2, 128, 64)
    S2 = _agg[64](hs2, src2d, dst2d)
    hs3 = _tc_mid(S2, hs2, dinv, b2.reshape(1, -1), W3, 64, 32)
    S3 = _agg[32](hs3, src2d, dst2d)
    outp, h3t = _tc_last(S3, hs3, dinv, b3.reshape(1, -1), Wc,
                         bc.reshape(1, -1))
    return outp[:N_NODES], h3t[:N_NODES]


# speedup vs baseline: 26.0409x; 2.3609x over previous
import functools

import jax
import jax.numpy as jnp
from jax import lax
from jax.experimental import pallas as pl
from jax.experimental.pallas import tpu as pltpu
from jax.experimental.pallas import tpu_sc as plsc

N_NODES = 10000
N_PAD = 10240
E = 320000
NC, NS = 2, 16
CHUNK = 128
SUP = 8
ROWS = 2560
E_PAD = ROWS * CHUNK
ROWS_PER_CORE = ROWS // NC
ROWS_PER_TILE = ROWS_PER_CORE // NS
OUTER = ROWS_PER_TILE // SUP
STRIPE = N_PAD // NS
RB = 1024
GRID = N_PAD // RB

_mesh = plsc.VectorSubcoreMesh(core_axis_name="c", subcore_axis_name="s")


def _zero_vmem_2d(ref, rows, cols):
    def row(i, carry):
        for c0 in range(0, cols, 16):
            ref[i, pl.ds(c0, 16)] = jnp.zeros((16,), jnp.float32)
        return carry
    lax.fori_loop(0, rows, row, 0)


@functools.partial(
    pl.kernel,
    out_type=jax.ShapeDtypeStruct((NC, N_PAD), jnp.float32),
    mesh=_mesh,
    scratch_types=[
        pltpu.VMEM_SHARED((N_PAD,), jnp.float32),
        pltpu.VMEM((SUP, CHUNK), jnp.int32),
        pltpu.VMEM((CHUNK,), jnp.float32),
        pltpu.VMEM((STRIPE,), jnp.float32),
    ],
)
def _sc_degree(dst_hbm, out_hbm, acc, didx, ones_v, zstripe):
    c = lax.axis_index("c")
    s = lax.axis_index("s")

    def fill(i, carry):
        ones_v[pl.ds(i * 16, 16)] = jnp.ones((16,), jnp.float32)
        return carry
    lax.fori_loop(0, CHUNK // 16, fill, 0)

    def zfill(i, carry):
        zstripe[pl.ds(i * 16, 16)] = jnp.zeros((16,), jnp.float32)
        return carry
    lax.fori_loop(0, STRIPE // 16, zfill, 0)
    pltpu.sync_copy(zstripe, acc.at[pl.ds(s * STRIPE, STRIPE)])
    plsc.subcore_barrier()

    base_row = c * ROWS_PER_CORE + s * ROWS_PER_TILE

    def outer(o, carry):
        pltpu.sync_copy(dst_hbm.at[pl.ds(base_row + o * SUP, SUP)], didx)
        for j in range(SUP):
            pltpu.sync_copy(ones_v, acc.at[didx.at[j]], add=True)
        return carry
    lax.fori_loop(0, OUTER, outer, 0)
    plsc.subcore_barrier()
    pltpu.sync_copy(acc.at[pl.ds(s * STRIPE, STRIPE)],
                    out_hbm.at[c, pl.ds(s * STRIPE, STRIPE)])


def _make_agg(C):
    @functools.partial(
        pl.kernel,
        out_type=jax.ShapeDtypeStruct((NC, N_PAD, C), jnp.float32),
        mesh=_mesh,
        scratch_types=[
            pltpu.VMEM_SHARED((N_PAD, C), jnp.float32),
            pltpu.VMEM((SUP, CHUNK), jnp.int32),
            pltpu.VMEM((SUP, CHUNK), jnp.int32),
            pltpu.VMEM((CHUNK, C), jnp.float32),
            pltpu.VMEM((CHUNK, C), jnp.float32),
            pltpu.SemaphoreType.DMA,
            pltpu.SemaphoreType.DMA,
        ],
        compiler_params=pltpu.CompilerParams(use_tc_tiling_on_sc=False),
    )
    def agg(table_hbm, src_hbm, dst_hbm, out_hbm,
            acc, sidx, didx, bufa, bufb, sema, semb):
        c = lax.axis_index("c")
        s = lax.axis_index("s")

        _zero_vmem_2d(bufa, CHUNK, C)
        for k in range(STRIPE // CHUNK):
            pltpu.sync_copy(bufa, acc.at[pl.ds(s * STRIPE + k * CHUNK, CHUNK)])
        plsc.subcore_barrier()

        base_row = c * ROWS_PER_CORE + s * ROWS_PER_TILE

        def outer(o, carry):
            r0 = base_row + o * SUP
            pltpu.sync_copy(src_hbm.at[pl.ds(r0, SUP)], sidx)
            pltpu.sync_copy(dst_hbm.at[pl.ds(r0, SUP)], didx)
            cp = pltpu.async_copy(table_hbm.at[sidx.at[0]], bufa, sema)
            for j in range(SUP):
                buf_cur = bufa if j % 2 == 0 else bufb
                buf_nxt = bufb if j % 2 == 0 else bufa
                cp.wait()
                if j + 1 < SUP:
                    sem_nxt = semb if j % 2 == 0 else sema
                    cp = pltpu.async_copy(
                        table_hbm.at[sidx.at[j + 1]], buf_nxt, sem_nxt)
                pltpu.sync_copy(buf_cur, acc.at[didx.at[j]], add=True)
            return carry
        lax.fori_loop(0, OUTER, outer, 0)
        plsc.subcore_barrier()
        for k in range(STRIPE // CHUNK):
            r = s * STRIPE + k * CHUNK
            pltpu.sync_copy(acc.at[pl.ds(r, CHUNK)],
                            out_hbm.at[c, pl.ds(r, CHUNK)])
    return agg


_agg = {C: _make_agg(C) for C in (128, 64, 32)}


def _tc_first_body(x_ref, w_ref, d0_ref, d1_ref, hs_ref, dinv_ref):
    deg = d0_ref[0] + d1_ref[0] + 1.0
    dinv = lax.rsqrt(deg)
    h = jnp.dot(x_ref[...], w_ref[...], preferred_element_type=jnp.float32)
    hs_ref[...] = dinv * h
    dinv_ref[...] = dinv


def _tc_first(xp, W1, D):
    return pl.pallas_call(
        _tc_first_body,
        grid=(GRID,),
        in_specs=[
            pl.BlockSpec((RB, 128), lambda i: (i, 0)),
            pl.BlockSpec((128, 128), lambda i: (0, 0)),
            pl.BlockSpec((1, RB, 1), lambda i: (0, i, 0)),
            pl.BlockSpec((1, RB, 1), lambda i: (1, i, 0)),
        ],
        out_specs=[
            pl.BlockSpec((RB, 128), lambda i: (i, 0)),
            pl.BlockSpec((RB, 1), lambda i: (i, 0)),
        ],
        out_shape=[
            jax.ShapeDtypeStruct((N_PAD, 128), jnp.float32),
            jax.ShapeDtypeStruct((N_PAD, 1), jnp.float32),
        ],
    )(xp, W1, D, D)


def _tc_mid_body(s0_ref, s1_ref, hs_ref, dinv_ref, b_ref, w_ref, o_ref):
    dinv = dinv_ref[...]
    z = dinv * (s0_ref[0] + s1_ref[0] + hs_ref[...]) + b_ref[...]
    ht = jnp.tanh(z)
    o_ref[...] = dinv * jnp.dot(ht, w_ref[...],
                                preferred_element_type=jnp.float32)


def _tc_mid(S, hs, dinv, b, W, Cin, Cout):
    return pl.pallas_call(
        _tc_mid_body,
        grid=(GRID,),
        in_specs=[
            pl.BlockSpec((1, RB, Cin), lambda i: (0, i, 0)),
            pl.BlockSpec((1, RB, Cin), lambda i: (1, i, 0)),
            pl.BlockSpec((RB, Cin), lambda i: (i, 0)),
            pl.BlockSpec((RB, 1), lambda i: (i, 0)),
            pl.BlockSpec((1, Cin), lambda i: (0, 0)),
            pl.BlockSpec((Cin, Cout), lambda i: (0, 0)),
        ],
        out_specs=pl.BlockSpec((RB, Cout), lambda i: (i, 0)),
        out_shape=jax.ShapeDtypeStruct((N_PAD, Cout), jnp.float32),
    )(S, S, hs, dinv, b, W)


def _tc_last_body(s0_ref, s1_ref, hs_ref, dinv_ref, b_ref, wc_ref, bc_ref,
                  out_ref, h_ref):
    dinv = dinv_ref[...]
    z = dinv * (s0_ref[0] + s1_ref[0] + hs_ref[...]) + b_ref[...]
    ht = jnp.tanh(z)
    h_ref[...] = ht
    out_ref[...] = (jnp.dot(ht, wc_ref[...],
                            preferred_element_type=jnp.float32) + bc_ref[...])


def _tc_last(S, hs, dinv, b3, Wc, bc):
    return pl.pallas_call(
        _tc_last_body,
        grid=(GRID,),
        in_specs=[
            pl.BlockSpec((1, RB, 32), lambda i: (0, i, 0)),
            pl.BlockSpec((1, RB, 32), lambda i: (1, i, 0)),
            pl.BlockSpec((RB, 32), lambda i: (i, 0)),
            pl.BlockSpec((RB, 1), lambda i: (i, 0)),
            pl.BlockSpec((1, 32), lambda i: (0, 0)),
            pl.BlockSpec((32, 40), lambda i: (0, 0)),
            pl.BlockSpec((1, 40), lambda i: (0, 0)),
        ],
        out_specs=[
            pl.BlockSpec((RB, 40), lambda i: (i, 0)),
            pl.BlockSpec((RB, 32), lambda i: (i, 0)),
        ],
        out_shape=[
            jax.ShapeDtypeStruct((N_PAD, 40), jnp.float32),
            jax.ShapeDtypeStruct((N_PAD, 32), jnp.float32),
        ],
    )(S, S, hs, dinv, b3, Wc, bc)


def kernel(x, edge_index, W1, b1, W2, b2, W3, b3, Wc, bc):
    src = edge_index[0].astype(jnp.int32)
    dst = edge_index[1].astype(jnp.int32)
    pad = N_NODES + jnp.arange(E_PAD - E, dtype=jnp.int32) % (N_PAD - N_NODES)
    src2d = jnp.concatenate([src, pad]).reshape(ROWS, CHUNK)
    dst2d = jnp.concatenate([dst, pad]).reshape(ROWS, CHUNK)
    xp = jnp.pad(x, ((0, N_PAD - N_NODES), (0, 0)))

    D = _sc_degree(dst2d).reshape(NC, N_PAD, 1)
    hs1, dinv = _tc_first(xp, W1, D)
    S1 = _agg[128](hs1, src2d, dst2d)
    hs2 = _tc_mid(S1, hs1, dinv, b1.reshape(1, -1), W2, 128, 64)
    S2 = _agg[64](hs2, src2d, dst2d)
    hs3 = _tc_mid(S2, hs2, dinv, b2.reshape(1, -1), W3, 64, 32)
    S3 = _agg[32](hs3, src2d, dst2d)
    outp, h3t = _tc_last(S3, hs3, dinv, b3.reshape(1, -1), Wc,
                         bc.reshape(1, -1))
    return outp[:N_NODES], h3t[:N_NODES]


# speedup vs baseline: 28.3101x; 1.0871x over previous
import functools

import jax
import jax.numpy as jnp
from jax import lax
from jax.experimental import pallas as pl
from jax.experimental.pallas import tpu as pltpu
from jax.experimental.pallas import tpu_sc as plsc

N_NODES = 10000
N_PAD = 10240
E = 320000
NC, NS = 2, 16
CHUNK = 128
SUP = 8
ROWS = 2560
E_PAD = ROWS * CHUNK
ROWS_PER_CORE = ROWS // NC
ROWS_PER_TILE = ROWS_PER_CORE // NS
OUTER = ROWS_PER_TILE // SUP
STRIPE = N_PAD // NS
RB = 1024
GRID = N_PAD // RB

_mesh = plsc.VectorSubcoreMesh(core_axis_name="c", subcore_axis_name="s")


def _zero_vmem_2d(ref, rows, cols):
    def row(i, carry):
        for c0 in range(0, cols, 16):
            ref[i, pl.ds(c0, 16)] = jnp.zeros((16,), jnp.float32)
        return carry
    lax.fori_loop(0, rows, row, 0)


@functools.partial(
    pl.kernel,
    out_type=jax.ShapeDtypeStruct((NC, N_PAD), jnp.float32),
    mesh=_mesh,
    scratch_types=[
        pltpu.VMEM_SHARED((N_PAD,), jnp.float32),
        pltpu.VMEM((SUP, CHUNK), jnp.int32),
        pltpu.VMEM((CHUNK,), jnp.float32),
        pltpu.VMEM((STRIPE,), jnp.float32),
    ],
)
def _sc_degree(dst_hbm, out_hbm, acc, didx, ones_v, zstripe):
    c = lax.axis_index("c")
    s = lax.axis_index("s")

    def fill(i, carry):
        ones_v[pl.ds(i * 16, 16)] = jnp.ones((16,), jnp.float32)
        return carry
    lax.fori_loop(0, CHUNK // 16, fill, 0)

    def zfill(i, carry):
        zstripe[pl.ds(i * 16, 16)] = jnp.zeros((16,), jnp.float32)
        return carry
    lax.fori_loop(0, STRIPE // 16, zfill, 0)
    pltpu.sync_copy(zstripe, acc.at[pl.ds(s * STRIPE, STRIPE)])
    plsc.subcore_barrier()

    base_row = c * ROWS_PER_CORE + s * ROWS_PER_TILE

    def outer(o, carry):
        pltpu.sync_copy(dst_hbm.at[pl.ds(base_row + o * SUP, SUP)], didx)
        for j in range(SUP):
            pltpu.sync_copy(ones_v, acc.at[didx.at[j]], add=True)
        return carry
    lax.fori_loop(0, OUTER, outer, 0)
    plsc.subcore_barrier()
    pltpu.sync_copy(acc.at[pl.ds(s * STRIPE, STRIPE)],
                    out_hbm.at[c, pl.ds(s * STRIPE, STRIPE)])


def _make_agg(C):
    @functools.partial(
        pl.kernel,
        out_type=jax.ShapeDtypeStruct((NC, N_PAD, C), jnp.float32),
        mesh=_mesh,
        scratch_types=[
            pltpu.VMEM_SHARED((N_PAD, C), jnp.float32),
            pltpu.VMEM((2, SUP, CHUNK), jnp.int32),
            pltpu.VMEM((2, SUP, CHUNK), jnp.int32),
            pltpu.VMEM((CHUNK, C), jnp.float32),
            pltpu.VMEM((CHUNK, C), jnp.float32),
            pltpu.SemaphoreType.DMA,
            pltpu.SemaphoreType.DMA,
            pltpu.SemaphoreType.DMA,
        ],
        compiler_params=pltpu.CompilerParams(use_tc_tiling_on_sc=False),
    )
    def agg(table_hbm, src_hbm, dst_hbm, out_hbm,
            acc, sidx, didx, bufa, bufb, sema, semb, semi):
        c = lax.axis_index("c")
        s = lax.axis_index("s")

        _zero_vmem_2d(bufa, CHUNK, C)
        for k in range(STRIPE // CHUNK):
            pltpu.sync_copy(bufa, acc.at[pl.ds(s * STRIPE + k * CHUNK, CHUNK)])
        plsc.subcore_barrier()

        base_row = c * ROWS_PER_CORE + s * ROWS_PER_TILE

        pltpu.sync_copy(src_hbm.at[pl.ds(base_row, SUP)], sidx.at[0])
        pltpu.sync_copy(dst_hbm.at[pl.ds(base_row, SUP)], didx.at[0])
        pltpu.async_copy(table_hbm.at[sidx.at[0, 0]], bufa, sema)

        def outer(o, carry):
            b = o % 2
            bn = (o + 1) % 2
            rn = base_row + (o + 1) * SUP
            pltpu.async_copy(src_hbm.at[pl.ds(rn, SUP)], sidx.at[bn], semi)
            pltpu.async_copy(dst_hbm.at[pl.ds(rn, SUP)], didx.at[bn], semi)
            for j in range(SUP):
                buf_cur = bufa if j % 2 == 0 else bufb
                sem_cur = sema if j % 2 == 0 else semb
                buf_nxt = bufb if j % 2 == 0 else bufa
                sem_nxt = semb if j % 2 == 0 else sema
                pltpu.make_async_copy(
                    table_hbm.at[sidx.at[b, j]], buf_cur, sem_cur).wait()
                if j + 1 < SUP:
                    pltpu.async_copy(
                        table_hbm.at[sidx.at[b, j + 1]], buf_nxt, sem_nxt)
                else:
                    pltpu.make_async_copy(
                        src_hbm.at[pl.ds(rn, SUP)], sidx.at[bn], semi).wait()
                    pltpu.make_async_copy(
                        dst_hbm.at[pl.ds(rn, SUP)], didx.at[bn], semi).wait()
                    pltpu.async_copy(
                        table_hbm.at[sidx.at[bn, 0]], buf_nxt, sem_nxt)
                pltpu.sync_copy(buf_cur, acc.at[didx.at[b, j]], add=True)
            return carry
        lax.fori_loop(0, OUTER, outer, 0)
        pltpu.make_async_copy(
            table_hbm.at[sidx.at[0, 0]], bufa, sema).wait()
        plsc.subcore_barrier()
        for k in range(STRIPE // CHUNK):
            r = s * STRIPE + k * CHUNK
            pltpu.sync_copy(acc.at[pl.ds(r, CHUNK)],
                            out_hbm.at[c, pl.ds(r, CHUNK)])
    return agg


_agg = {C: _make_agg(C) for C in (128, 64, 32)}


def _tc_first_body(x_ref, w_ref, d0_ref, d1_ref, hs_ref, dinv_ref):
    deg = d0_ref[0] + d1_ref[0] + 1.0
    dinv = lax.rsqrt(deg)
    h = jnp.dot(x_ref[...], w_ref[...], preferred_element_type=jnp.float32)
    hs_ref[...] = dinv * h
    dinv_ref[...] = dinv


def _tc_first(xp, W1, D):
    return pl.pallas_call(
        _tc_first_body,
        grid=(GRID,),
        in_specs=[
            pl.BlockSpec((RB, 128), lambda i: (i, 0)),
            pl.BlockSpec((128, 128), lambda i: (0, 0)),
            pl.BlockSpec((1, RB, 1), lambda i: (0, i, 0)),
            pl.BlockSpec((1, RB, 1), lambda i: (1, i, 0)),
        ],
        out_specs=[
            pl.BlockSpec((RB, 128), lambda i: (i, 0)),
            pl.BlockSpec((RB, 1), lambda i: (i, 0)),
        ],
        out_shape=[
            jax.ShapeDtypeStruct((N_PAD, 128), jnp.float32),
            jax.ShapeDtypeStruct((N_PAD, 1), jnp.float32),
        ],
    )(xp, W1, D, D)


def _tc_mid_body(s0_ref, s1_ref, hs_ref, dinv_ref, b_ref, w_ref, o_ref):
    dinv = dinv_ref[...]
    z = dinv * (s0_ref[0] + s1_ref[0] + hs_ref[...]) + b_ref[...]
    ht = jnp.tanh(z)
    o_ref[...] = dinv * jnp.dot(ht, w_ref[...],
                                preferred_element_type=jnp.float32)


def _tc_mid(S, hs, dinv, b, W, Cin, Cout):
    return pl.pallas_call(
        _tc_mid_body,
        grid=(GRID,),
        in_specs=[
            pl.BlockSpec((1, RB, Cin), lambda i: (0, i, 0)),
            pl.BlockSpec((1, RB, Cin), lambda i: (1, i, 0)),
            pl.BlockSpec((RB, Cin), lambda i: (i, 0)),
            pl.BlockSpec((RB, 1), lambda i: (i, 0)),
            pl.BlockSpec((1, Cin), lambda i: (0, 0)),
            pl.BlockSpec((Cin, Cout), lambda i: (0, 0)),
        ],
        out_specs=pl.BlockSpec((RB, Cout), lambda i: (i, 0)),
        out_shape=jax.ShapeDtypeStruct((N_PAD, Cout), jnp.float32),
    )(S, S, hs, dinv, b, W)


def _tc_last_body(s0_ref, s1_ref, hs_ref, dinv_ref, b_ref, wc_ref, bc_ref,
                  out_ref, h_ref):
    dinv = dinv_ref[...]
    z = dinv * (s0_ref[0] + s1_ref[0] + hs_ref[...]) + b_ref[...]
    ht = jnp.tanh(z)
    h_ref[...] = ht
    out_ref[...] = (jnp.dot(ht, wc_ref[...],
                            preferred_element_type=jnp.float32) + bc_ref[...])


def _tc_last(S, hs, dinv, b3, Wc, bc):
    return pl.pallas_call(
        _tc_last_body,
        grid=(GRID,),
        in_specs=[
            pl.BlockSpec((1, RB, 32), lambda i: (0, i, 0)),
            pl.BlockSpec((1, RB, 32), lambda i: (1, i, 0)),
            pl.BlockSpec((RB, 32), lambda i: (i, 0)),
            pl.BlockSpec((RB, 1), lambda i: (i, 0)),
            pl.BlockSpec((1, 32), lambda i: (0, 0)),
            pl.BlockSpec((32, 40), lambda i: (0, 0)),
            pl.BlockSpec((1, 40), lambda i: (0, 0)),
        ],
        out_specs=[
            pl.BlockSpec((RB, 40), lambda i: (i, 0)),
            pl.BlockSpec((RB, 32), lambda i: (i, 0)),
        ],
        out_shape=[
            jax.ShapeDtypeStruct((N_PAD, 40), jnp.float32),
            jax.ShapeDtypeStruct((N_PAD, 32), jnp.float32),
        ],
    )(S, S, hs, dinv, b3, Wc, bc)


def kernel(x, edge_index, W1, b1, W2, b2, W3, b3, Wc, bc):
    src = edge_index[0].astype(jnp.int32)
    dst = edge_index[1].astype(jnp.int32)
    n_extra = E_PAD - E + SUP * CHUNK
    pad = N_NODES + jnp.arange(n_extra, dtype=jnp.int32) % (N_PAD - N_NODES)
    src2d = jnp.concatenate([src, pad]).reshape(ROWS + SUP, CHUNK)
    dst2d = jnp.concatenate([dst, pad]).reshape(ROWS + SUP, CHUNK)
    xp = jnp.pad(x, ((0, N_PAD - N_NODES), (0, 0)))

    D = _sc_degree(dst2d).reshape(NC, N_PAD, 1)
    hs1, dinv = _tc_first(xp, W1, D)
    S1 = _agg[128](hs1, src2d, dst2d)
    hs2 = _tc_mid(S1, hs1, dinv, b1.reshape(1, -1), W2, 128, 64)
    S2 = _agg[64](hs2, src2d, dst2d)
    hs3 = _tc_mid(S2, hs2, dinv, b2.reshape(1, -1), W3, 64, 32)
    S3 = _agg[32](hs3, src2d, dst2d)
    outp, h3t = _tc_last(S3, hs3, dinv, b3.reshape(1, -1), Wc,
                         bc.reshape(1, -1))
    return outp[:N_NODES], h3t[:N_NODES]


# speedup vs baseline: 32.6875x; 1.1546x over previous
import functools

import jax
import jax.numpy as jnp
from jax import lax
from jax.experimental import pallas as pl
from jax.experimental.pallas import tpu as pltpu
from jax.experimental.pallas import tpu_sc as plsc

N_NODES = 10000
N_PAD = 10240
E = 320000
NC, NS = 2, 16
CHUNK = 128
SUP = 8
ROWS = 2560
E_PAD = ROWS * CHUNK
ROWS_PER_CORE = ROWS // NC
ROWS_PER_TILE = ROWS_PER_CORE // NS
OUTER = ROWS_PER_TILE // SUP
STRIPE = N_PAD // NS
RB = 1024
GRID = N_PAD // RB

_mesh = plsc.VectorSubcoreMesh(core_axis_name="c", subcore_axis_name="s")


def _zero_vmem_2d(ref, rows, cols):
    def row(i, carry):
        for c0 in range(0, cols, 16):
            ref[i, pl.ds(c0, 16)] = jnp.zeros((16,), jnp.float32)
        return carry
    lax.fori_loop(0, rows, row, 0)


@functools.partial(
    pl.kernel,
    out_type=jax.ShapeDtypeStruct((NC, N_PAD), jnp.float32),
    mesh=_mesh,
    scratch_types=[
        pltpu.VMEM_SHARED((N_PAD,), jnp.float32),
        pltpu.VMEM((SUP, CHUNK), jnp.int32),
        pltpu.VMEM((CHUNK,), jnp.float32),
        pltpu.VMEM((STRIPE,), jnp.float32),
    ],
)
def _sc_degree(dst_hbm, out_hbm, acc, didx, ones_v, zstripe):
    c = lax.axis_index("c")
    s = lax.axis_index("s")

    def fill(i, carry):
        ones_v[pl.ds(i * 16, 16)] = jnp.ones((16,), jnp.float32)
        return carry
    lax.fori_loop(0, CHUNK // 16, fill, 0)

    def zfill(i, carry):
        zstripe[pl.ds(i * 16, 16)] = jnp.zeros((16,), jnp.float32)
        return carry
    lax.fori_loop(0, STRIPE // 16, zfill, 0)
    pltpu.sync_copy(zstripe, acc.at[pl.ds(s * STRIPE, STRIPE)])
    plsc.subcore_barrier()

    base_row = c * ROWS_PER_CORE + s * ROWS_PER_TILE

    def outer(o, carry):
        pltpu.sync_copy(dst_hbm.at[pl.ds(base_row + o * SUP, SUP)], didx)
        for j in range(SUP):
            pltpu.sync_copy(ones_v, acc.at[didx.at[j]], add=True)
        return carry
    lax.fori_loop(0, OUTER, outer, 0)
    plsc.subcore_barrier()
    pltpu.sync_copy(acc.at[pl.ds(s * STRIPE, STRIPE)],
                    out_hbm.at[c, pl.ds(s * STRIPE, STRIPE)])


def _make_agg(C):
    rpg = 128 // C
    n_inner = SUP // rpg
    rpgc = rpg * CHUNK
    spt = ROWS_PER_TILE // rpg
    spc = ROWS_PER_CORE // rpg

    @functools.partial(
        pl.kernel,
        out_type=jax.ShapeDtypeStruct((NC, N_PAD, C), jnp.float32),
        mesh=_mesh,
        scratch_types=[
            pltpu.VMEM_SHARED((N_PAD, C), jnp.float32),
            pltpu.VMEM((2, n_inner, 1, rpgc), jnp.int32),
            pltpu.VMEM((2, n_inner, 1, rpgc), jnp.int32),
            pltpu.VMEM((rpgc, C), jnp.float32),
            pltpu.VMEM((rpgc, C), jnp.float32),
            pltpu.SemaphoreType.DMA,
            pltpu.SemaphoreType.DMA,
            pltpu.SemaphoreType.DMA,
        ],
        compiler_params=pltpu.CompilerParams(use_tc_tiling_on_sc=False),
    )
    def agg(table_hbm, src_hbm, dst_hbm, out_hbm,
            acc, sidx, didx, bufa, bufb, sema, semb, semi):
        c = lax.axis_index("c")
        s = lax.axis_index("s")

        _zero_vmem_2d(bufa, CHUNK, C)
        for k in range(STRIPE // CHUNK):
            pltpu.sync_copy(
                bufa.at[pl.ds(0, CHUNK)],
                acc.at[pl.ds(s * STRIPE + k * CHUNK, CHUNK)])
        plsc.subcore_barrier()

        base = c * spc + s * spt

        pltpu.sync_copy(src_hbm.at[pl.ds(base, n_inner)], sidx.at[0])
        pltpu.sync_copy(dst_hbm.at[pl.ds(base, n_inner)], didx.at[0])
        pltpu.async_copy(table_hbm.at[sidx.at[0, 0, 0]], bufa, sema)

        def outer(o, carry):
            b = o % 2
            bn = (o + 1) % 2
            rn = base + (o + 1) * n_inner
            pltpu.async_copy(src_hbm.at[pl.ds(rn, n_inner)], sidx.at[bn],
                             semi)
            pltpu.async_copy(dst_hbm.at[pl.ds(rn, n_inner)], didx.at[bn],
                             semi)
            for j in range(n_inner):
                buf_cur = bufa if j % 2 == 0 else bufb
                sem_cur = sema if j % 2 == 0 else semb
                buf_nxt = bufb if j % 2 == 0 else bufa
                sem_nxt = semb if j % 2 == 0 else sema
                pltpu.make_async_copy(
                    table_hbm.at[sidx.at[b, j, 0]], buf_cur, sem_cur).wait()
                if j + 1 < n_inner:
                    pltpu.async_copy(
                        table_hbm.at[sidx.at[b, j + 1, 0]], buf_nxt, sem_nxt)
                else:
                    pltpu.make_async_copy(
                        src_hbm.at[pl.ds(rn, n_inner)], sidx.at[bn],
                        semi).wait()
                    pltpu.make_async_copy(
                        dst_hbm.at[pl.ds(rn, n_inner)], didx.at[bn],
                        semi).wait()
                    pltpu.async_copy(
                        table_hbm.at[sidx.at[bn, 0, 0]], buf_nxt, sem_nxt)
                pltpu.sync_copy(buf_cur, acc.at[didx.at[b, j, 0]], add=True)
            return carry
        lax.fori_loop(0, OUTER, outer, 0)
        pltpu.make_async_copy(
            table_hbm.at[sidx.at[0, 0, 0]], bufa, sema).wait()
        plsc.subcore_barrier()
        for k in range(STRIPE // CHUNK):
            r = s * STRIPE + k * CHUNK
            pltpu.sync_copy(acc.at[pl.ds(r, CHUNK)],
                            out_hbm.at[c, pl.ds(r, CHUNK)])
    return agg


_agg = {C: _make_agg(C) for C in (128, 64, 32)}


def _tc_first_body(x_ref, w_ref, d0_ref, d1_ref, hs_ref, dinv_ref):
    deg = d0_ref[0] + d1_ref[0] + 1.0
    dinv = lax.rsqrt(deg)
    h = jnp.dot(x_ref[...], w_ref[...], preferred_element_type=jnp.float32)
    hs_ref[...] = dinv * h
    dinv_ref[...] = dinv


def _tc_first(xp, W1, D):
    return pl.pallas_call(
        _tc_first_body,
        grid=(GRID,),
        in_specs=[
            pl.BlockSpec((RB, 128), lambda i: (i, 0)),
            pl.BlockSpec((128, 128), lambda i: (0, 0)),
            pl.BlockSpec((1, RB, 1), lambda i: (0, i, 0)),
            pl.BlockSpec((1, RB, 1), lambda i: (1, i, 0)),
        ],
        out_specs=[
            pl.BlockSpec((RB, 128), lambda i: (i, 0)),
            pl.BlockSpec((RB, 1), lambda i: (i, 0)),
        ],
        out_shape=[
            jax.ShapeDtypeStruct((N_PAD, 128), jnp.float32),
            jax.ShapeDtypeStruct((N_PAD, 1), jnp.float32),
        ],
    )(xp, W1, D, D)


def _tc_mid_body(s0_ref, s1_ref, hs_ref, dinv_ref, b_ref, w_ref, o_ref):
    dinv = dinv_ref[...]
    z = dinv * (s0_ref[0] + s1_ref[0] + hs_ref[...]) + b_ref[...]
    ht = jnp.tanh(z)
    o_ref[...] = dinv * jnp.dot(ht, w_ref[...],
                                preferred_element_type=jnp.float32)


def _tc_mid(S, hs, dinv, b, W, Cin, Cout):
    return pl.pallas_call(
        _tc_mid_body,
        grid=(GRID,),
        in_specs=[
            pl.BlockSpec((1, RB, Cin), lambda i: (0, i, 0)),
            pl.BlockSpec((1, RB, Cin), lambda i: (1, i, 0)),
            pl.BlockSpec((RB, Cin), lambda i: (i, 0)),
            pl.BlockSpec((RB, 1), lambda i: (i, 0)),
            pl.BlockSpec((1, Cin), lambda i: (0, 0)),
            pl.BlockSpec((Cin, Cout), lambda i: (0, 0)),
        ],
        out_specs=pl.BlockSpec((RB, Cout), lambda i: (i, 0)),
        out_shape=jax.ShapeDtypeStruct((N_PAD, Cout), jnp.float32),
    )(S, S, hs, dinv, b, W)


def _tc_last_body(s0_ref, s1_ref, hs_ref, dinv_ref, b_ref, wc_ref, bc_ref,
                  out_ref, h_ref):
    dinv = dinv_ref[...]
    z = dinv * (s0_ref[0] + s1_ref[0] + hs_ref[...]) + b_ref[...]
    ht = jnp.tanh(z)
    h_ref[...] = ht
    out_ref[...] = (jnp.dot(ht, wc_ref[...],
                            preferred_element_type=jnp.float32) + bc_ref[...])


def _tc_last(S, hs, dinv, b3, Wc, bc):
    rb = N_NODES // GRID
    return pl.pallas_call(
        _tc_last_body,
        grid=(GRID,),
        in_specs=[
            pl.BlockSpec((1, rb, 32), lambda i: (0, i, 0)),
            pl.BlockSpec((1, rb, 32), lambda i: (1, i, 0)),
            pl.BlockSpec((rb, 32), lambda i: (i, 0)),
            pl.BlockSpec((rb, 1), lambda i: (i, 0)),
            pl.BlockSpec((1, 32), lambda i: (0, 0)),
            pl.BlockSpec((32, 40), lambda i: (0, 0)),
            pl.BlockSpec((1, 40), lambda i: (0, 0)),
        ],
        out_specs=[
            pl.BlockSpec((rb, 40), lambda i: (i, 0)),
            pl.BlockSpec((rb, 32), lambda i: (i, 0)),
        ],
        out_shape=[
            jax.ShapeDtypeStruct((N_NODES, 40), jnp.float32),
            jax.ShapeDtypeStruct((N_NODES, 32), jnp.float32),
        ],
    )(S, S, hs, dinv, b3, Wc, bc)


def kernel(x, edge_index, W1, b1, W2, b2, W3, b3, Wc, bc):
    src = edge_index[0].astype(jnp.int32)
    dst = edge_index[1].astype(jnp.int32)
    n_extra = E_PAD - E + SUP * CHUNK
    pad = N_NODES + jnp.arange(n_extra, dtype=jnp.int32) % (N_PAD - N_NODES)
    src_flat = jnp.concatenate([src, pad])
    dst_flat = jnp.concatenate([dst, pad])
    dst2d = dst_flat.reshape(ROWS + SUP, CHUNK)

    def views(rpg):
        w = rpg * CHUNK
        return src_flat.reshape(-1, 1, w), dst_flat.reshape(-1, 1, w)
    xp = jnp.pad(x, ((0, N_PAD - N_NODES), (0, 0)))

    D = _sc_degree(dst2d).reshape(NC, N_PAD, 1)
    hs1, dinv = _tc_first(xp, W1, D)
    s1v, d1v = views(1)
    S1 = _agg[128](hs1, s1v, d1v)
    hs2 = _tc_mid(S1, hs1, dinv, b1.reshape(1, -1), W2, 128, 64)
    s2v, d2v = views(2)
    S2 = _agg[64](hs2, s2v, d2v)
    hs3 = _tc_mid(S2, hs2, dinv, b2.reshape(1, -1), W3, 64, 32)
    s3v, d3v = views(4)
    S3 = _agg[32](hs3, s3v, d3v)
    outp, h3t = _tc_last(S3, hs3, dinv, b3.reshape(1, -1), Wc,
                         bc.reshape(1, -1))
    return outp, h3t


# speedup vs baseline: 35.5854x; 1.0887x over previous
import functools

import jax
import jax.numpy as jnp
from jax import lax
from jax.experimental import pallas as pl
from jax.experimental.pallas import tpu as pltpu
from jax.experimental.pallas import tpu_sc as plsc

N_NODES = 10000
N_PAD = 10240
E = 320000
NC, NS = 2, 16
CHUNK = 128
SUP = 8
ROWS = 2560
E_PAD = ROWS * CHUNK
ROWS_PER_CORE = ROWS // NC
ROWS_PER_TILE = ROWS_PER_CORE // NS
OUTER = ROWS_PER_TILE // SUP
STRIPE = N_PAD // NS
RB = 1024
GRID = N_PAD // RB

_mesh = plsc.VectorSubcoreMesh(core_axis_name="c", subcore_axis_name="s")


def _zero_vmem_2d(ref, rows, cols):
    def row(i, carry):
        for c0 in range(0, cols, 16):
            ref[i, pl.ds(c0, 16)] = jnp.zeros((16,), jnp.float32)
        return carry
    lax.fori_loop(0, rows, row, 0)


@functools.partial(
    pl.kernel,
    out_type=jax.ShapeDtypeStruct((NC, N_PAD), jnp.float32),
    mesh=_mesh,
    scratch_types=[
        pltpu.VMEM_SHARED((N_PAD,), jnp.float32),
        pltpu.VMEM((SUP, CHUNK), jnp.int32),
        pltpu.VMEM((CHUNK,), jnp.float32),
        pltpu.VMEM((STRIPE,), jnp.float32),
    ],
)
def _sc_degree(dst_hbm, out_hbm, acc, didx, ones_v, zstripe):
    c = lax.axis_index("c")
    s = lax.axis_index("s")

    def fill(i, carry):
        ones_v[pl.ds(i * 16, 16)] = jnp.ones((16,), jnp.float32)
        return carry
    lax.fori_loop(0, CHUNK // 16, fill, 0)

    def zfill(i, carry):
        zstripe[pl.ds(i * 16, 16)] = jnp.zeros((16,), jnp.float32)
        return carry
    lax.fori_loop(0, STRIPE // 16, zfill, 0)
    pltpu.sync_copy(zstripe, acc.at[pl.ds(s * STRIPE, STRIPE)])
    plsc.subcore_barrier()

    base_row = c * ROWS_PER_CORE + s * ROWS_PER_TILE

    def outer(o, carry):
        pltpu.sync_copy(dst_hbm.at[pl.ds(base_row + o * SUP, SUP)], didx)
        for j in range(SUP):
            pltpu.sync_copy(ones_v, acc.at[didx.at[j]], add=True)
        return carry
    lax.fori_loop(0, OUTER, outer, 0)
    plsc.subcore_barrier()
    pltpu.sync_copy(acc.at[pl.ds(s * STRIPE, STRIPE)],
                    out_hbm.at[c, pl.ds(s * STRIPE, STRIPE)])


def _make_agg_deep(C, rpg, rpgc, spt, spc):
    NB = 4

    @functools.partial(
        pl.kernel,
        out_type=jax.ShapeDtypeStruct((NC, N_PAD, C), jnp.float32),
        mesh=_mesh,
        scratch_types=[
            pltpu.VMEM_SHARED((N_PAD, C), jnp.float32),
            pltpu.VMEM((spt, 1, rpgc), jnp.int32),
            pltpu.VMEM((spt, 1, rpgc), jnp.int32),
            [pltpu.VMEM((rpgc, C), jnp.float32) for _ in range(NB)],
            [pltpu.SemaphoreType.DMA for _ in range(NB)],
            [pltpu.SemaphoreType.DMA for _ in range(NB)],
        ],
        compiler_params=pltpu.CompilerParams(use_tc_tiling_on_sc=False),
    )
    def agg(table_hbm, src_hbm, dst_hbm, out_hbm,
            acc, sidx, didx, bufs, gsem, ssem):
        c = lax.axis_index("c")
        s = lax.axis_index("s")
        base = c * spc + s * spt

        pltpu.sync_copy(src_hbm.at[pl.ds(base, spt)], sidx)
        pltpu.sync_copy(dst_hbm.at[pl.ds(base, spt)], didx)
        for k in range(NB - 1):
            pltpu.async_copy(table_hbm.at[sidx.at[k, 0]], bufs[k], gsem[k])

        _zero_vmem_2d(bufs[NB - 1], CHUNK, C)
        for k in range(STRIPE // CHUNK):
            pltpu.sync_copy(
                bufs[NB - 1].at[pl.ds(0, CHUNK)],
                acc.at[pl.ds(s * STRIPE + k * CHUNK, CHUNK)])
        plsc.subcore_barrier()

        def body(q, carry):
            for r in range(NB):
                t = q * NB + r
                pltpu.make_async_copy(
                    table_hbm.at[sidx.at[0, 0]], bufs[r], gsem[r]).wait()
                pltpu.async_copy(bufs[r], acc.at[didx.at[t, 0]], ssem[r],
                                 add=True)
                rp = (r + NB - 1) % NB
                if r == 0:
                    @pl.when(q >= 1)
                    def _():
                        pltpu.make_async_copy(
                            bufs[rp], acc.at[didx.at[0, 0]], ssem[rp]).wait()
                else:
                    pltpu.make_async_copy(
                        bufs[r - 1], acc.at[didx.at[0, 0]],
                        ssem[r - 1]).wait()

                @pl.when(t + NB - 1 < spt)
                def _():
                    pltpu.async_copy(table_hbm.at[sidx.at[t + NB - 1, 0]],
                                     bufs[rp], gsem[rp])
            return carry
        lax.fori_loop(0, spt // NB, body, 0)
        rl = (spt - 1) % NB
        pltpu.make_async_copy(
            bufs[rl], acc.at[didx.at[0, 0]], ssem[rl]).wait()
        plsc.subcore_barrier()
        for k in range(STRIPE // CHUNK):
            r = s * STRIPE + k * CHUNK
            pltpu.sync_copy(acc.at[pl.ds(r, CHUNK)],
                            out_hbm.at[c, pl.ds(r, CHUNK)])
    return agg


def _make_agg(C):
    rpg = 128 // C
    n_inner = SUP // rpg
    rpgc = rpg * CHUNK
    spt = ROWS_PER_TILE // rpg
    spc = ROWS_PER_CORE // rpg
    if rpg > 1:
        return _make_agg_deep(C, rpg, rpgc, spt, spc)

    @functools.partial(
        pl.kernel,
        out_type=jax.ShapeDtypeStruct((NC, N_PAD, C), jnp.float32),
        mesh=_mesh,
        scratch_types=[
            pltpu.VMEM_SHARED((N_PAD, C), jnp.float32),
            pltpu.VMEM((2, n_inner, 1, rpgc), jnp.int32),
            pltpu.VMEM((2, n_inner, 1, rpgc), jnp.int32),
            pltpu.VMEM((rpgc, C), jnp.float32),
            pltpu.VMEM((rpgc, C), jnp.float32),
            pltpu.SemaphoreType.DMA,
            pltpu.SemaphoreType.DMA,
            pltpu.SemaphoreType.DMA,
        ],
        compiler_params=pltpu.CompilerParams(use_tc_tiling_on_sc=False),
    )
    def agg(table_hbm, src_hbm, dst_hbm, out_hbm,
            acc, sidx, didx, bufa, bufb, sema, semb, semi):
        c = lax.axis_index("c")
        s = lax.axis_index("s")

        base = c * spc + s * spt

        pltpu.sync_copy(src_hbm.at[pl.ds(base, n_inner)], sidx.at[0])
        pltpu.sync_copy(dst_hbm.at[pl.ds(base, n_inner)], didx.at[0])
        pltpu.async_copy(table_hbm.at[sidx.at[0, 0, 0]], bufa, sema)

        _zero_vmem_2d(bufb, CHUNK, C)
        for k in range(STRIPE // CHUNK):
            pltpu.sync_copy(
                bufb.at[pl.ds(0, CHUNK)],
                acc.at[pl.ds(s * STRIPE + k * CHUNK, CHUNK)])
        plsc.subcore_barrier()

        def outer(o, carry):
            b = o % 2
            bn = (o + 1) % 2
            rn = base + (o + 1) * n_inner
            pltpu.async_copy(src_hbm.at[pl.ds(rn, n_inner)], sidx.at[bn],
                             semi)
            pltpu.async_copy(dst_hbm.at[pl.ds(rn, n_inner)], didx.at[bn],
                             semi)
            for j in range(n_inner):
                buf_cur = bufa if j % 2 == 0 else bufb
                sem_cur = sema if j % 2 == 0 else semb
                buf_nxt = bufb if j % 2 == 0 else bufa
                sem_nxt = semb if j % 2 == 0 else sema
                pltpu.make_async_copy(
                    table_hbm.at[sidx.at[b, j, 0]], buf_cur, sem_cur).wait()
                if j + 1 < n_inner:
                    pltpu.async_copy(
                        table_hbm.at[sidx.at[b, j + 1, 0]], buf_nxt, sem_nxt)
                else:
                    pltpu.make_async_copy(
                        src_hbm.at[pl.ds(rn, n_inner)], sidx.at[bn],
                        semi).wait()
                    pltpu.make_async_copy(
                        dst_hbm.at[pl.ds(rn, n_inner)], didx.at[bn],
                        semi).wait()
                    pltpu.async_copy(
                        table_hbm.at[sidx.at[bn, 0, 0]], buf_nxt, sem_nxt)
                pltpu.sync_copy(buf_cur, acc.at[didx.at[b, j, 0]], add=True)
            return carry
        lax.fori_loop(0, OUTER, outer, 0)
        pltpu.make_async_copy(
            table_hbm.at[sidx.at[0, 0, 0]], bufa, sema).wait()
        plsc.subcore_barrier()
        for k in range(STRIPE // CHUNK):
            r = s * STRIPE + k * CHUNK
            pltpu.sync_copy(acc.at[pl.ds(r, CHUNK)],
                            out_hbm.at[c, pl.ds(r, CHUNK)])
    return agg


_agg = {C: _make_agg(C) for C in (128, 64, 32)}


def _tc_first_body(x_ref, w_ref, d0_ref, d1_ref, hs_ref, dinv_ref):
    deg = d0_ref[0] + d1_ref[0] + 1.0
    dinv = lax.rsqrt(deg)
    h = jnp.dot(x_ref[...], w_ref[...], preferred_element_type=jnp.float32)
    hs_ref[...] = dinv * h
    dinv_ref[...] = dinv


def _tc_first(xp, W1, D):
    return pl.pallas_call(
        _tc_first_body,
        grid=(GRID,),
        in_specs=[
            pl.BlockSpec((RB, 128), lambda i: (i, 0)),
            pl.BlockSpec((128, 128), lambda i: (0, 0)),
            pl.BlockSpec((1, RB, 1), lambda i: (0, i, 0)),
            pl.BlockSpec((1, RB, 1), lambda i: (1, i, 0)),
        ],
        out_specs=[
            pl.BlockSpec((RB, 128), lambda i: (i, 0)),
            pl.BlockSpec((RB, 1), lambda i: (i, 0)),
        ],
        out_shape=[
            jax.ShapeDtypeStruct((N_PAD, 128), jnp.float32),
            jax.ShapeDtypeStruct((N_PAD, 1), jnp.float32),
        ],
    )(xp, W1, D, D)


def _tc_mid_body(s0_ref, s1_ref, hs_ref, dinv_ref, b_ref, w_ref, o_ref):
    dinv = dinv_ref[...]
    z = dinv * (s0_ref[0] + s1_ref[0] + hs_ref[...]) + b_ref[...]
    ht = jnp.tanh(z)
    o_ref[...] = dinv * jnp.dot(ht, w_ref[...],
                                preferred_element_type=jnp.float32)


def _tc_mid(S, hs, dinv, b, W, Cin, Cout):
    return pl.pallas_call(
        _tc_mid_body,
        grid=(GRID,),
        in_specs=[
            pl.BlockSpec((1, RB, Cin), lambda i: (0, i, 0)),
            pl.BlockSpec((1, RB, Cin), lambda i: (1, i, 0)),
            pl.BlockSpec((RB, Cin), lambda i: (i, 0)),
            pl.BlockSpec((RB, 1), lambda i: (i, 0)),
            pl.BlockSpec((1, Cin), lambda i: (0, 0)),
            pl.BlockSpec((Cin, Cout), lambda i: (0, 0)),
        ],
        out_specs=pl.BlockSpec((RB, Cout), lambda i: (i, 0)),
        out_shape=jax.ShapeDtypeStruct((N_PAD, Cout), jnp.float32),
    )(S, S, hs, dinv, b, W)


def _tc_last_body(s0_ref, s1_ref, hs_ref, dinv_ref, b_ref, wc_ref, bc_ref,
                  out_ref, h_ref):
    dinv = dinv_ref[...]
    z = dinv * (s0_ref[0] + s1_ref[0] + hs_ref[...]) + b_ref[...]
    ht = jnp.tanh(z)
    h_ref[...] = ht
    out_ref[...] = (jnp.dot(ht, wc_ref[...],
                            preferred_element_type=jnp.float32) + bc_ref[...])


def _tc_last(S, hs, dinv, b3, Wc, bc):
    rb = N_NODES // GRID
    return pl.pallas_call(
        _tc_last_body,
        grid=(GRID,),
        in_specs=[
            pl.BlockSpec((1, rb, 32), lambda i: (0, i, 0)),
            pl.BlockSpec((1, rb, 32), lambda i: (1, i, 0)),
            pl.BlockSpec((rb, 32), lambda i: (i, 0)),
            pl.BlockSpec((rb, 1), lambda i: (i, 0)),
            pl.BlockSpec((1, 32), lambda i: (0, 0)),
            pl.BlockSpec((32, 40), lambda i: (0, 0)),
            pl.BlockSpec((1, 40), lambda i: (0, 0)),
        ],
        out_specs=[
            pl.BlockSpec((rb, 40), lambda i: (i, 0)),
            pl.BlockSpec((rb, 32), lambda i: (i, 0)),
        ],
        out_shape=[
            jax.ShapeDtypeStruct((N_NODES, 40), jnp.float32),
            jax.ShapeDtypeStruct((N_NODES, 32), jnp.float32),
        ],
    )(S, S, hs, dinv, b3, Wc, bc)


def kernel(x, edge_index, W1, b1, W2, b2, W3, b3, Wc, bc):
    src = edge_index[0].astype(jnp.int32)
    dst = edge_index[1].astype(jnp.int32)
    n_extra = E_PAD - E + SUP * CHUNK
    pad = N_NODES + jnp.arange(n_extra, dtype=jnp.int32) % (N_PAD - N_NODES)
    src_flat = jnp.concatenate([src, pad])
    dst_flat = jnp.concatenate([dst, pad])
    dst2d = dst_flat.reshape(ROWS + SUP, CHUNK)

    def views(rpg):
        w = rpg * CHUNK
        return src_flat.reshape(-1, 1, w), dst_flat.reshape(-1, 1, w)
    xp = jnp.pad(x, ((0, N_PAD - N_NODES), (0, 0)))

    D = _sc_degree(dst2d).reshape(NC, N_PAD, 1)
    hs1, dinv = _tc_first(xp, W1, D)
    s1v, d1v = views(1)
    S1 = _agg[128](hs1, s1v, d1v)
    hs2 = _tc_mid(S1, hs1, dinv, b1.reshape(1, -1), W2, 128, 64)
    s2v, d2v = views(2)
    S2 = _agg[64](hs2, s2v, d2v)
    hs3 = _tc_mid(S2, hs2, dinv, b2.reshape(1, -1), W3, 64, 32)
    s3v, d3v = views(4)
    S3 = _agg[32](hs3, s3v, d3v)
    outp, h3t = _tc_last(S3, hs3, dinv, b3.reshape(1, -1), Wc,
                         bc.reshape(1, -1))
    return outp, h3t


# speedup vs baseline: 35.6096x; 1.0007x over previous
import functools

import jax
import jax.numpy as jnp
from jax import lax
from jax.experimental import pallas as pl
from jax.experimental.pallas import tpu as pltpu
from jax.experimental.pallas import tpu_sc as plsc

N_NODES = 10000
N_PAD = 10240
E = 320000
NC, NS = 2, 16
CHUNK = 128
SUP = 8
ROWS = 2560
E_PAD = ROWS * CHUNK
ROWS_PER_CORE = ROWS // NC
ROWS_PER_TILE = ROWS_PER_CORE // NS
OUTER = ROWS_PER_TILE // SUP
STRIPE = N_PAD // NS
RB = 1024
GRID = N_PAD // RB

_mesh = plsc.VectorSubcoreMesh(core_axis_name="c", subcore_axis_name="s")


def _zero_vmem_2d(ref, rows, cols):
    def row(i, carry):
        for c0 in range(0, cols, 16):
            ref[i, pl.ds(c0, 16)] = jnp.zeros((16,), jnp.float32)
        return carry
    lax.fori_loop(0, rows, row, 0)


_DEG_W = SUP * CHUNK
_DEG_SPT = ROWS_PER_TILE // SUP
_DEG_SPC = ROWS_PER_CORE // SUP


@functools.partial(
    pl.kernel,
    out_type=jax.ShapeDtypeStruct((NC, N_PAD), jnp.float32),
    mesh=_mesh,
    scratch_types=[
        pltpu.VMEM_SHARED((N_PAD,), jnp.float32),
        pltpu.VMEM((_DEG_SPT, 1, _DEG_W), jnp.int32),
        pltpu.VMEM((_DEG_W,), jnp.float32),
        pltpu.VMEM((STRIPE,), jnp.float32),
    ],
)
def _sc_degree(dst_hbm, out_hbm, acc, didx, ones_v, zstripe):
    c = lax.axis_index("c")
    s = lax.axis_index("s")
    base = c * _DEG_SPC + s * _DEG_SPT
    pltpu.sync_copy(dst_hbm.at[pl.ds(base, _DEG_SPT)], didx)

    def fill(i, carry):
        ones_v[pl.ds(i * 16, 16)] = jnp.ones((16,), jnp.float32)
        return carry
    lax.fori_loop(0, _DEG_W // 16, fill, 0)

    def zfill(i, carry):
        zstripe[pl.ds(i * 16, 16)] = jnp.zeros((16,), jnp.float32)
        return carry
    lax.fori_loop(0, STRIPE // 16, zfill, 0)
    pltpu.sync_copy(zstripe, acc.at[pl.ds(s * STRIPE, STRIPE)])
    plsc.subcore_barrier()

    def outer(t, carry):
        pltpu.sync_copy(ones_v, acc.at[didx.at[t, 0]], add=True)
        return carry
    lax.fori_loop(0, _DEG_SPT, outer, 0)
    plsc.subcore_barrier()
    pltpu.sync_copy(acc.at[pl.ds(s * STRIPE, STRIPE)],
                    out_hbm.at[c, pl.ds(s * STRIPE, STRIPE)])


def _make_agg_deep(C, rpg, rpgc, spt, spc):
    NB = 4

    @functools.partial(
        pl.kernel,
        out_type=jax.ShapeDtypeStruct((NC, N_PAD, C), jnp.float32),
        mesh=_mesh,
        scratch_types=[
            pltpu.VMEM_SHARED((N_PAD, C), jnp.float32),
            pltpu.VMEM((spt, 1, rpgc), jnp.int32),
            pltpu.VMEM((spt, 1, rpgc), jnp.int32),
            [pltpu.VMEM((rpgc, C), jnp.float32) for _ in range(NB)],
            [pltpu.SemaphoreType.DMA for _ in range(NB)],
            [pltpu.SemaphoreType.DMA for _ in range(NB)],
        ],
        compiler_params=pltpu.CompilerParams(use_tc_tiling_on_sc=False),
    )
    def agg(table_hbm, src_hbm, dst_hbm, out_hbm,
            acc, sidx, didx, bufs, gsem, ssem):
        c = lax.axis_index("c")
        s = lax.axis_index("s")
        base = c * spc + s * spt

        pltpu.sync_copy(src_hbm.at[pl.ds(base, spt)], sidx)
        pltpu.sync_copy(dst_hbm.at[pl.ds(base, spt)], didx)
        for k in range(NB - 1):
            pltpu.async_copy(table_hbm.at[sidx.at[k, 0]], bufs[k], gsem[k])

        _zero_vmem_2d(bufs[NB - 1], rpgc, C)
        off = 0
        while off < STRIPE:
            blk = min(rpgc, STRIPE - off)
            pltpu.sync_copy(
                bufs[NB - 1].at[pl.ds(0, blk)],
                acc.at[pl.ds(s * STRIPE + off, blk)])
            off += blk
        plsc.subcore_barrier()

        def body(q, carry):
            for r in range(NB):
                t = q * NB + r
                pltpu.make_async_copy(
                    table_hbm.at[sidx.at[0, 0]], bufs[r], gsem[r]).wait()
                pltpu.async_copy(bufs[r], acc.at[didx.at[t, 0]], ssem[r],
                                 add=True)
                rp = (r + NB - 1) % NB
                if r == 0:
                    @pl.when(q >= 1)
                    def _():
                        pltpu.make_async_copy(
                            bufs[rp], acc.at[didx.at[0, 0]], ssem[rp]).wait()
                else:
                    pltpu.make_async_copy(
                        bufs[r - 1], acc.at[didx.at[0, 0]],
                        ssem[r - 1]).wait()

                @pl.when(t + NB - 1 < spt)
                def _():
                    pltpu.async_copy(table_hbm.at[sidx.at[t + NB - 1, 0]],
                                     bufs[rp], gsem[rp])
            return carry
        lax.fori_loop(0, spt // NB, body, 0)
        rl = (spt - 1) % NB
        pltpu.make_async_copy(
            bufs[rl], acc.at[didx.at[0, 0]], ssem[rl]).wait()
        plsc.subcore_barrier()
        pltpu.sync_copy(acc.at[pl.ds(s * STRIPE, STRIPE)],
                        out_hbm.at[c, pl.ds(s * STRIPE, STRIPE)])
    return agg


def _make_agg(C):
    rpg = 128 // C
    n_inner = SUP // rpg
    rpgc = rpg * CHUNK
    spt = ROWS_PER_TILE // rpg
    spc = ROWS_PER_CORE // rpg
    if rpg > 1:
        return _make_agg_deep(C, rpg, rpgc, spt, spc)

    @functools.partial(
        pl.kernel,
        out_type=jax.ShapeDtypeStruct((NC, N_PAD, C), jnp.float32),
        mesh=_mesh,
        scratch_types=[
            pltpu.VMEM_SHARED((N_PAD, C), jnp.float32),
            pltpu.VMEM((2, n_inner, 1, rpgc), jnp.int32),
            pltpu.VMEM((2, n_inner, 1, rpgc), jnp.int32),
            pltpu.VMEM((rpgc, C), jnp.float32),
            pltpu.VMEM((rpgc, C), jnp.float32),
            pltpu.SemaphoreType.DMA,
            pltpu.SemaphoreType.DMA,
            pltpu.SemaphoreType.DMA,
        ],
        compiler_params=pltpu.CompilerParams(use_tc_tiling_on_sc=False),
    )
    def agg(table_hbm, src_hbm, dst_hbm, out_hbm,
            acc, sidx, didx, bufa, bufb, sema, semb, semi):
        c = lax.axis_index("c")
        s = lax.axis_index("s")

        base = c * spc + s * spt

        pltpu.sync_copy(src_hbm.at[pl.ds(base, n_inner)], sidx.at[0])
        pltpu.sync_copy(dst_hbm.at[pl.ds(base, n_inner)], didx.at[0])
        pltpu.async_copy(table_hbm.at[sidx.at[0, 0, 0]], bufa, sema)

        _zero_vmem_2d(bufb, CHUNK, C)
        for k in range(STRIPE // CHUNK):
            pltpu.sync_copy(
                bufb.at[pl.ds(0, CHUNK)],
                acc.at[pl.ds(s * STRIPE + k * CHUNK, CHUNK)])
        plsc.subcore_barrier()

        def outer(o, carry):
            b = o % 2
            bn = (o + 1) % 2
            rn = base + (o + 1) * n_inner
            pltpu.async_copy(src_hbm.at[pl.ds(rn, n_inner)], sidx.at[bn],
                             semi)
            pltpu.async_copy(dst_hbm.at[pl.ds(rn, n_inner)], didx.at[bn],
                             semi)
            for j in range(n_inner):
                buf_cur = bufa if j % 2 == 0 else bufb
                sem_cur = sema if j % 2 == 0 else semb
                buf_nxt = bufb if j % 2 == 0 else bufa
                sem_nxt = semb if j % 2 == 0 else sema
                pltpu.make_async_copy(
                    table_hbm.at[sidx.at[b, j, 0]], buf_cur, sem_cur).wait()
                if j + 1 < n_inner:
                    pltpu.async_copy(
                        table_hbm.at[sidx.at[b, j + 1, 0]], buf_nxt, sem_nxt)
                else:
                    pltpu.make_async_copy(
                        src_hbm.at[pl.ds(rn, n_inner)], sidx.at[bn],
                        semi).wait()
                    pltpu.make_async_copy(
                        dst_hbm.at[pl.ds(rn, n_inner)], didx.at[bn],
                        semi).wait()
                    pltpu.async_copy(
                        table_hbm.at[sidx.at[bn, 0, 0]], buf_nxt, sem_nxt)
                pltpu.sync_copy(buf_cur, acc.at[didx.at[b, j, 0]], add=True)
            return carry
        lax.fori_loop(0, OUTER, outer, 0)
        pltpu.make_async_copy(
            table_hbm.at[sidx.at[0, 0, 0]], bufa, sema).wait()
        plsc.subcore_barrier()
        pltpu.sync_copy(acc.at[pl.ds(s * STRIPE, STRIPE)],
                        out_hbm.at[c, pl.ds(s * STRIPE, STRIPE)])
    return agg


_agg = {C: _make_agg(C) for C in (128, 64, 32)}


def _tc_first_body(x_ref, w_ref, d0_ref, d1_ref, hs_ref, dinv_ref):
    deg = d0_ref[0] + d1_ref[0] + 1.0
    dinv = lax.rsqrt(deg)
    h = jnp.dot(x_ref[...], w_ref[...], preferred_element_type=jnp.float32)
    hs_ref[...] = dinv * h
    dinv_ref[...] = dinv


def _tc_first(xp, W1, D):
    return pl.pallas_call(
        _tc_first_body,
        grid=(GRID,),
        in_specs=[
            pl.BlockSpec((RB, 128), lambda i: (i, 0)),
            pl.BlockSpec((128, 128), lambda i: (0, 0)),
            pl.BlockSpec((1, RB, 1), lambda i: (0, i, 0)),
            pl.BlockSpec((1, RB, 1), lambda i: (1, i, 0)),
        ],
        out_specs=[
            pl.BlockSpec((RB, 128), lambda i: (i, 0)),
            pl.BlockSpec((RB, 1), lambda i: (i, 0)),
        ],
        out_shape=[
            jax.ShapeDtypeStruct((N_PAD, 128), jnp.float32),
            jax.ShapeDtypeStruct((N_PAD, 1), jnp.float32),
        ],
    )(xp, W1, D, D)


def _tc_mid_body(s0_ref, s1_ref, hs_ref, dinv_ref, b_ref, w_ref, o_ref):
    dinv = dinv_ref[...]
    z = dinv * (s0_ref[0] + s1_ref[0] + hs_ref[...]) + b_ref[...]
    ht = jnp.tanh(z)
    o_ref[...] = dinv * jnp.dot(ht, w_ref[...],
                                preferred_element_type=jnp.float32)


def _tc_mid(S, hs, dinv, b, W, Cin, Cout):
    return pl.pallas_call(
        _tc_mid_body,
        grid=(GRID,),
        in_specs=[
            pl.BlockSpec((1, RB, Cin), lambda i: (0, i, 0)),
            pl.BlockSpec((1, RB, Cin), lambda i: (1, i, 0)),
            pl.BlockSpec((RB, Cin), lambda i: (i, 0)),
            pl.BlockSpec((RB, 1), lambda i: (i, 0)),
            pl.BlockSpec((1, Cin), lambda i: (0, 0)),
            pl.BlockSpec((Cin, Cout), lambda i: (0, 0)),
        ],
        out_specs=pl.BlockSpec((RB, Cout), lambda i: (i, 0)),
        out_shape=jax.ShapeDtypeStruct((N_PAD, Cout), jnp.float32),
    )(S, S, hs, dinv, b, W)


def _tc_last_body(s0_ref, s1_ref, hs_ref, dinv_ref, b_ref, wc_ref, bc_ref,
                  out_ref, h_ref):
    dinv = dinv_ref[...]
    z = dinv * (s0_ref[0] + s1_ref[0] + hs_ref[...]) + b_ref[...]
    ht = jnp.tanh(z)
    h_ref[...] = ht
    out_ref[...] = (jnp.dot(ht, wc_ref[...],
                            preferred_element_type=jnp.float32) + bc_ref[...])


def _tc_last(S, hs, dinv, b3, Wc, bc):
    rb = N_NODES // GRID
    return pl.pallas_call(
        _tc_last_body,
        grid=(GRID,),
        in_specs=[
            pl.BlockSpec((1, rb, 32), lambda i: (0, i, 0)),
            pl.BlockSpec((1, rb, 32), lambda i: (1, i, 0)),
            pl.BlockSpec((rb, 32), lambda i: (i, 0)),
            pl.BlockSpec((rb, 1), lambda i: (i, 0)),
            pl.BlockSpec((1, 32), lambda i: (0, 0)),
            pl.BlockSpec((32, 40), lambda i: (0, 0)),
            pl.BlockSpec((1, 40), lambda i: (0, 0)),
        ],
        out_specs=[
            pl.BlockSpec((rb, 40), lambda i: (i, 0)),
            pl.BlockSpec((rb, 32), lambda i: (i, 0)),
        ],
        out_shape=[
            jax.ShapeDtypeStruct((N_NODES, 40), jnp.float32),
            jax.ShapeDtypeStruct((N_NODES, 32), jnp.float32),
        ],
    )(S, S, hs, dinv, b3, Wc, bc)


def kernel(x, edge_index, W1, b1, W2, b2, W3, b3, Wc, bc):
    src = edge_index[0].astype(jnp.int32)
    dst = edge_index[1].astype(jnp.int32)
    n_extra = E_PAD - E + SUP * CHUNK
    pad = N_NODES + jnp.arange(n_extra, dtype=jnp.int32) % (N_PAD - N_NODES)
    src_flat = jnp.concatenate([src, pad])
    dst_flat = jnp.concatenate([dst, pad])
    dst_deg = dst_flat.reshape(-1, 1, _DEG_W)

    def views(rpg):
        w = rpg * CHUNK
        return src_flat.reshape(-1, 1, w), dst_flat.reshape(-1, 1, w)
    xp = jnp.pad(x, ((0, N_PAD - N_NODES), (0, 0)))

    D = _sc_degree(dst_deg).reshape(NC, N_PAD, 1)
    hs1, dinv = _tc_first(xp, W1, D)
    s1v, d1v = views(1)
    S1 = _agg[128](hs1, s1v, d1v)
    hs2 = _tc_mid(S1, hs1, dinv, b1.reshape(1, -1), W2, 128, 64)
    s2v, d2v = views(2)
    S2 = _agg[64](hs2, s2v, d2v)
    hs3 = _tc_mid(S2, hs2, dinv, b2.reshape(1, -1), W3, 64, 32)
    s3v, d3v = views(4)
    S3 = _agg[32](hs3, s3v, d3v)
    outp, h3t = _tc_last(S3, hs3, dinv, b3.reshape(1, -1), Wc,
                         bc.reshape(1, -1))
    return outp, h3t
